# Initial kernel scaffold; baseline (speedup 1.0000x reference)
import jax
import jax.numpy as jnp
from jax.experimental import pallas as pl


def kernel(inputs):
    raise NotImplementedError("write your pallas kernel here")


# speedup vs baseline: 31.6331x; 31.6331x over previous
import functools

import jax
import jax.numpy as jnp
from jax import lax
from jax.experimental import pallas as pl
from jax.experimental.pallas import tpu as pltpu
from jax.experimental.pallas import tpu_sc as plsc

B, S, C = 32, 4096, 256
K = 8
L = 16
CG = C // L
SH = S // 2
W1 = 8
N1 = SH // W1
W2 = 8
N2 = N1 // W2
NEG = float("-inf")


def _insert_v(t, v):
    t = list(t)
    for i in range(K):
        hi = jnp.maximum(t[i], v)
        v = jnp.minimum(t[i], v)
        t[i] = hi
    return tuple(t)


def _insert_iv(t, ti, v, iv):
    t, ti = list(t), list(ti)
    for i in range(K):
        c = v > t[i]
        t_new = jnp.where(c, v, t[i])
        v_new = jnp.where(c, t[i], v)
        ti_new = jnp.where(c, iv, ti[i])
        iv = jnp.where(c, ti[i], iv)
        t[i], ti[i], v = t_new, ti_new, v_new
    return tuple(t), tuple(ti)


def _sc_body(x_hbm, out_hbm, buf_a, buf_b, cm, sm, gbuf, cbuf, rbuf, obuf,
             sem_a, sem_b):
    nc = plsc.get_sparse_core_info().num_cores
    b = lax.axis_index("s") * nc + lax.axis_index("c")
    iota = lax.iota(jnp.int32, L)
    zero = jnp.zeros((L,), jnp.int32)
    neg = jnp.full((L,), NEG, jnp.float32)

    def dma(cg, h, buf, sem):
        return pltpu.make_async_copy(
            x_hbm.at[b, pl.ds(h * SH, SH), pl.ds(cg * L, L)], buf, sem)

    def process(buf, cg, h):
        def p1(ch, _):
            base = ch * W1
            m = buf[base]
            for r in range(1, W1):
                m = jnp.maximum(m, buf[base + r])
            cm[ch] = m
            return 0

        lax.fori_loop(0, N1, p1, 0, unroll=False)

        def p2(g, _):
            base = g * W2
            m = cm[base]
            for r in range(1, W2):
                m = jnp.maximum(m, cm[base + r])
            sm[g] = m
            return 0

        lax.fori_loop(0, N2, p2, 0, unroll=False)

        def p3(j, carry):
            t, ti = carry
            return _insert_iv(t, ti, sm[j], zero + j)

        t, ti = lax.fori_loop(0, N2, p3, ((neg,) * K, (zero,) * K),
                              unroll=False)
        for i in range(K):
            gbuf[i] = ti[i]

        def p4(j, carry):
            t, ti = carry
            idx = gbuf[j // W2] * W2 + (j % W2)
            v = plsc.load_gather(cm, [idx, iota])
            return _insert_iv(t, ti, v, idx)

        t, ti = lax.fori_loop(0, K * W2, p4, ((neg,) * K, (zero,) * K),
                              unroll=False)
        for i in range(K):
            cbuf[i] = ti[i]

        def p5(j, t):
            row = cbuf[j // W1] * W1 + (j % W1)
            return _insert_v(t, plsc.load_gather(buf, [row, iota]))

        init = (neg,) * K if h == 0 else tuple(rbuf[i] for i in range(K))
        t = lax.fori_loop(0, K * W1, p5, init, unroll=False)

        if h == 0:
            for i in range(K):
                rbuf[i] = t[i]
        else:
            base = cg * (L * K)
            for r in range(K):
                plsc.store_scatter(obuf, [iota * K + (base + r)], t[r])

    dma(0, 0, buf_a, sem_a).start()

    def cg_body(cg, _):
        dma(cg, 1, buf_b, sem_b).start()
        dma(cg, 0, buf_a, sem_a).wait()
        process(buf_a, cg, 0)

        @pl.when(cg + 1 < CG)
        def _():
            dma(cg + 1, 0, buf_a, sem_a).start()

        dma(cg, 1, buf_b, sem_b).wait()
        process(buf_b, cg, 1)
        return 0

    lax.fori_loop(0, CG, cg_body, 0, unroll=False)
    pltpu.sync_copy(obuf, out_hbm.at[b])


@jax.jit
def kernel(inputs):
    x = inputs
    mesh = plsc.VectorSubcoreMesh(core_axis_name="c", subcore_axis_name="s")
    run = pl.kernel(
        _sc_body,
        out_type=jax.ShapeDtypeStruct((B, C * K), jnp.float32),
        mesh=mesh,
        compiler_params=pltpu.CompilerParams(
            use_tc_tiling_on_sc=False, needs_layout_passes=False),
        scratch_types=[
            pltpu.VMEM((SH, L), jnp.float32),
            pltpu.VMEM((SH, L), jnp.float32),
            pltpu.VMEM((N1, L), jnp.float32),
            pltpu.VMEM((N2, L), jnp.float32),
            pltpu.VMEM((K, L), jnp.int32),
            pltpu.VMEM((K, L), jnp.int32),
            pltpu.VMEM((K, L), jnp.float32),
            pltpu.VMEM((C * K,), jnp.float32),
            pltpu.SemaphoreType.DMA,
            pltpu.SemaphoreType.DMA,
        ],
    )
    return run(x)


# speedup vs baseline: 37.8141x; 1.1954x over previous
import functools

import jax
import jax.numpy as jnp
from jax import lax
from jax.experimental import pallas as pl
from jax.experimental.pallas import tpu as pltpu
from jax.experimental.pallas import tpu_sc as plsc

B, S, C = 32, 4096, 256
K = 8
L = 16
CG = C // L
SH = S // 2
W1 = 8
N1 = SH // W1
W2 = 8
N2 = N1 // W2
NEG = float("-inf")


def _insert_v(t, v):
    t = list(t)
    for i in range(K):
        hi = jnp.maximum(t[i], v)
        v = jnp.minimum(t[i], v)
        t[i] = hi
    return tuple(t)


def _insert_iv(t, ti, v, iv):
    t, ti = list(t), list(ti)
    for i in range(K):
        c = v > t[i]
        t_new = jnp.where(c, v, t[i])
        v_new = jnp.where(c, t[i], v)
        ti_new = jnp.where(c, iv, ti[i])
        iv = jnp.where(c, ti[i], iv)
        t[i], ti[i], v = t_new, ti_new, v_new
    return tuple(t), tuple(ti)


def _sc_body(x_hbm, out_hbm, buf_a, buf_b, cm, sm, gbuf, cbuf, rbuf, obuf,
             sem_a, sem_b):
    nc = plsc.get_sparse_core_info().num_cores
    b = lax.axis_index("s") * nc + lax.axis_index("c")
    iota = lax.iota(jnp.int32, L)
    zero = jnp.zeros((L,), jnp.int32)
    neg = jnp.full((L,), NEG, jnp.float32)

    def dma(cg, h, buf, sem):
        return pltpu.make_async_copy(
            x_hbm.at[b, pl.ds(h * N1, N1), cg // W1, :,
                     pl.ds((cg % W1) * L, L)], buf, sem)

    def process(buf, cg, h):
        def p1(ch, _):
            m = buf[ch, 0]
            for r in range(1, W1):
                m = jnp.maximum(m, buf[ch, r])
            cm[ch] = m
            return 0

        lax.fori_loop(0, N1, p1, 0, unroll=False)

        def p2(g, _):
            base = g * W2
            m = cm[base]
            for r in range(1, W2):
                m = jnp.maximum(m, cm[base + r])
            sm[g] = m
            return 0

        lax.fori_loop(0, N2, p2, 0, unroll=False)

        def p3(j, carry):
            t, ti = carry
            return _insert_iv(t, ti, sm[j], zero + j)

        t, ti = lax.fori_loop(0, N2, p3, ((neg,) * K, (zero,) * K),
                              unroll=False)
        for i in range(K):
            gbuf[i] = ti[i]

        def p4(j, carry):
            t, ti = carry
            idx = gbuf[j // W2] * W2 + (j % W2)
            v = plsc.load_gather(cm, [idx, iota])
            return _insert_iv(t, ti, v, idx)

        t, ti = lax.fori_loop(0, K * W2, p4, ((neg,) * K, (zero,) * K),
                              unroll=False)
        for i in range(K):
            cbuf[i] = ti[i]

        def p5(j, t):
            chv = cbuf[j // W1]
            v = plsc.load_gather(buf, [chv, zero + (j % W1), iota])
            return _insert_v(t, v)

        init = (neg,) * K if h == 0 else tuple(rbuf[i] for i in range(K))
        t = lax.fori_loop(0, K * W1, p5, init, unroll=False)

        if h == 0:
            for i in range(K):
                rbuf[i] = t[i]
        else:
            base = cg * (L * K)
            for r in range(K):
                plsc.store_scatter(obuf, [iota * K + (base + r)], t[r])

    dma(0, 0, buf_a, sem_a).start()

    def cg_body(cg, _):
        dma(cg, 1, buf_b, sem_b).start()
        dma(cg, 0, buf_a, sem_a).wait()
        process(buf_a, cg, 0)

        @pl.when(cg + 1 < CG)
        def _():
            dma(cg + 1, 0, buf_a, sem_a).start()

        dma(cg, 1, buf_b, sem_b).wait()
        process(buf_b, cg, 1)
        return 0

    lax.fori_loop(0, CG, cg_body, 0, unroll=False)
    pltpu.sync_copy(obuf, out_hbm.at[b])


@jax.jit
def kernel(inputs):
    x = inputs
    x5 = jnp.transpose(x.reshape(B, S // 8, 8, C // 128, 128), (0, 1, 3, 2, 4))
    mesh = plsc.VectorSubcoreMesh(core_axis_name="c", subcore_axis_name="s")
    run = pl.kernel(
        _sc_body,
        out_type=jax.ShapeDtypeStruct((B, C * K), jnp.float32),
        mesh=mesh,
        compiler_params=pltpu.CompilerParams(
            use_tc_tiling_on_sc=False, needs_layout_passes=False),
        scratch_types=[
            pltpu.VMEM((N1, W1, L), jnp.float32),
            pltpu.VMEM((N1, W1, L), jnp.float32),
            pltpu.VMEM((N1, L), jnp.float32),
            pltpu.VMEM((N2, L), jnp.float32),
            pltpu.VMEM((K, L), jnp.int32),
            pltpu.VMEM((K, L), jnp.int32),
            pltpu.VMEM((K, L), jnp.float32),
            pltpu.VMEM((C * K,), jnp.float32),
            pltpu.SemaphoreType.DMA,
            pltpu.SemaphoreType.DMA,
        ],
    )
    return run(x5)


# speedup vs baseline: 45.6587x; 1.2075x over previous
import functools

import jax
import jax.numpy as jnp
from jax import lax
from jax.experimental import pallas as pl
from jax.experimental.pallas import tpu as pltpu
from jax.experimental.pallas import tpu_sc as plsc

B, S, C = 32, 4096, 256
K = 8
L = 16
M = 2 * S
NCO = 128 // L
QR = 2048
NQ = M // QR
NCH = QR // 8
N1 = NCH // 2
W2 = 4
N2 = N1 // W2
NEG = float("-inf")


def _insert_v(t, v):
    t = list(t)
    for i in range(K):
        hi = jnp.maximum(t[i], v)
        v = jnp.minimum(t[i], v)
        t[i] = hi
    return tuple(t)


def _insert_iv(t, ti, v, iv):
    t, ti = list(t), list(ti)
    for i in range(K):
        c = v > t[i]
        t_new = jnp.where(c, v, t[i])
        v_new = jnp.where(c, t[i], v)
        ti_new = jnp.where(c, iv, ti[i])
        iv = jnp.where(c, ti[i], iv)
        t[i], ti[i], v = t_new, ti_new, v_new
    return tuple(t), tuple(ti)


def _sc_body(x_hbm, out_hbm, buf_a, buf_b, cm, sm, gbuf, cbuf, rbuf, obuf,
             sem_a, sem_b):
    nc = plsc.get_sparse_core_info().num_cores
    b = lax.axis_index("s") * nc + lax.axis_index("c")
    iota = lax.iota(jnp.int32, L)
    zero = jnp.zeros((L,), jnp.int32)
    neg = jnp.full((L,), NEG, jnp.float32)

    def dma(co, q, buf, sem):
        return pltpu.make_async_copy(
            x_hbm.at[b, pl.ds(q * QR, QR), pl.ds(co * L, L)], buf, sem)

    def process(buf, co, q):
        def p1(i, _):
            for u in range(2):
                base = (i * 2 + u) * 8
                m = buf[base]
                for r in range(1, 8):
                    m = jnp.maximum(m, buf[base + r])
                cm[i * 2 + u] = m
            return 0

        lax.fori_loop(0, NCH // 2, p1, 0, unroll=False)

        for cset in range(2):
            def p2(g, _):
                m = cm[cset + 2 * (g * W2)]
                for r in range(1, W2):
                    m = jnp.maximum(m, cm[cset + 2 * (g * W2 + r)])
                sm[g] = m
                return 0

            lax.fori_loop(0, N2, p2, 0, unroll=False)

            def p3(j, carry):
                t, ti = carry
                return _insert_iv(t, ti, sm[j], zero + j)

            t, ti = lax.fori_loop(0, N2, p3, ((neg,) * K, (zero,) * K),
                                  unroll=False)
            for i in range(K):
                gbuf[i] = ti[i]

            def p4(j, carry):
                t, ti = carry
                idx = gbuf[j // W2] * W2 + (j % W2)
                v = plsc.load_gather(cm, [idx * 2 + cset, iota])
                return _insert_iv(t, ti, v, idx)

            t, ti = lax.fori_loop(0, K * W2, p4, ((neg,) * K, (zero,) * K),
                                  unroll=False)
            for i in range(K):
                cbuf[i] = ti[i]

            def p5(j, t):
                row = (cbuf[j // 8] * 2 + cset) * 8 + (j % 8)
                return _insert_v(t, plsc.load_gather(buf, [row, iota]))

            if q == 0:
                init = (neg,) * K
            else:
                init = tuple(rbuf[cset * K + i] for i in range(K))
            t = lax.fori_loop(0, K * 8, p5, init, unroll=False)

            if q == NQ - 1:
                base = cset * (128 * K) + co * (L * K)
                for r in range(K):
                    plsc.store_scatter(obuf, [iota * K + (base + r)], t[r])
            else:
                for i in range(K):
                    rbuf[cset * K + i] = t[i]

    dma(0, 0, buf_a, sem_a).start()

    def co_body(co, _):
        dma(co, 1, buf_b, sem_b).start()
        dma(co, 0, buf_a, sem_a).wait()
        process(buf_a, co, 0)
        dma(co, 2, buf_a, sem_a).start()
        dma(co, 1, buf_b, sem_b).wait()
        process(buf_b, co, 1)
        dma(co, 3, buf_b, sem_b).start()
        dma(co, 2, buf_a, sem_a).wait()
        process(buf_a, co, 2)

        @pl.when(co + 1 < NCO)
        def _():
            dma(co + 1, 0, buf_a, sem_a).start()

        dma(co, 3, buf_b, sem_b).wait()
        process(buf_b, co, 3)
        return 0

    lax.fori_loop(0, NCO, co_body, 0, unroll=False)
    pltpu.sync_copy(obuf, out_hbm.at[b])


@jax.jit
def kernel(inputs):
    x = inputs
    x6 = jnp.transpose(x.reshape(B, S // 8, 8, 2, 128),
                       (0, 1, 3, 2, 4)).reshape(B, M, 128)
    mesh = plsc.VectorSubcoreMesh(core_axis_name="c", subcore_axis_name="s")
    run = pl.kernel(
        _sc_body,
        out_type=jax.ShapeDtypeStruct((B, C * K), jnp.float32),
        mesh=mesh,
        compiler_params=pltpu.CompilerParams(
            use_tc_tiling_on_sc=False, needs_layout_passes=False),
        scratch_types=[
            pltpu.VMEM((QR, L), jnp.float32),
            pltpu.VMEM((QR, L), jnp.float32),
            pltpu.VMEM((NCH, L), jnp.float32),
            pltpu.VMEM((N2, L), jnp.float32),
            pltpu.VMEM((K, L), jnp.int32),
            pltpu.VMEM((K, L), jnp.int32),
            pltpu.VMEM((2 * K, L), jnp.float32),
            pltpu.VMEM((C * K,), jnp.float32),
            pltpu.SemaphoreType.DMA,
            pltpu.SemaphoreType.DMA,
        ],
    )
    return run(x6)


# speedup vs baseline: 49.7792x; 1.0902x over previous
import functools

import jax
import jax.numpy as jnp
from jax import lax
from jax.experimental import pallas as pl
from jax.experimental.pallas import tpu as pltpu
from jax.experimental.pallas import tpu_sc as plsc

B, S, C = 32, 4096, 256
K = 8
L = 16
M = 2 * S
NCO = 128 // L
QR = 2048
NQ = M // QR
NCH = QR // 8
N1 = NCH // 2
W2 = 4
N2 = N1 // W2
NEG = float("-inf")


def _insert_v(t, v):
    t = list(t)
    for i in range(K):
        hi = jnp.maximum(t[i], v)
        v = jnp.minimum(t[i], v)
        t[i] = hi
    return tuple(t)


def _insert_iv(t, ti, v, iv):
    t, ti = list(t), list(ti)
    for i in range(K):
        c = v > t[i]
        t_new = jnp.where(c, v, t[i])
        v_new = jnp.where(c, t[i], v)
        ti_new = jnp.where(c, iv, ti[i])
        iv = jnp.where(c, ti[i], iv)
        t[i], ti[i], v = t_new, ti_new, v_new
    return tuple(t), tuple(ti)


def _sc_body(x_hbm, out_hbm, buf_a, buf_b, cm, sm, gbuf, cbuf, rbuf, obuf,
             sem_a, sem_b):
    nc = plsc.get_sparse_core_info().num_cores
    b = lax.axis_index("s") * nc + lax.axis_index("c")
    iota = lax.iota(jnp.int32, L)
    zero = jnp.zeros((L,), jnp.int32)
    neg = jnp.full((L,), NEG, jnp.float32)

    def dma(co, q, buf, sem):
        return pltpu.make_async_copy(
            x_hbm.at[b, pl.ds(q * QR, QR), pl.ds(co * L, L)], buf, sem)

    def process(buf, co, q):
        def p12(g, _):
            acc = [None, None]
            for u in range(8):
                base = (g * 8 + u) * 8
                m = buf[base]
                for r in range(1, 8):
                    m = jnp.maximum(m, buf[base + r])
                cm[g * 8 + u] = m
                p = u % 2
                acc[p] = m if acc[p] is None else jnp.maximum(acc[p], m)
            sm[g] = acc[0]
            sm[N2 + g] = acc[1]
            return 0

        lax.fori_loop(0, N2, p12, 0, unroll=False)

        for cset in range(2):
            def p3(i, carry):
                t, ti = carry
                j = i * 2
                t, ti = _insert_iv(t, ti, sm[cset * N2 + j], zero + j)
                return _insert_iv(t, ti, sm[cset * N2 + j + 1], zero + (j + 1))

            t, ti = lax.fori_loop(0, N2 // 2, p3, ((neg,) * K, (zero,) * K),
                                  unroll=False)
            for i in range(K):
                gbuf[i] = ti[i]

            def p4(jg, carry):
                t, ti = carry
                grow = gbuf[jg]
                for u in range(W2):
                    idx = grow * W2 + u
                    v = plsc.load_gather(cm, [grow * (2 * W2) + (2 * u + cset),
                                              iota])
                    t, ti = _insert_iv(t, ti, v, idx)
                return t, ti

            t, ti = lax.fori_loop(0, K, p4, ((neg,) * K, (zero,) * K),
                                  unroll=False)
            for i in range(K):
                cbuf[i] = ti[i]

            def p5(jc, t):
                crow = cbuf[jc] * 16
                for r in range(8):
                    v = plsc.load_gather(buf, [crow + (cset * 8 + r), iota])
                    t = _insert_v(t, v)
                return t

            if q == 0:
                init = (neg,) * K
            else:
                init = tuple(rbuf[cset * K + i] for i in range(K))
            t = lax.fori_loop(0, K, p5, init, unroll=False)

            if q == NQ - 1:
                base = cset * (128 * K) + co * (L * K)
                for r in range(K):
                    plsc.store_scatter(obuf, [iota * K + (base + r)], t[r])
            else:
                for i in range(K):
                    rbuf[cset * K + i] = t[i]

    dma(0, 0, buf_a, sem_a).start()

    def co_body(co, _):
        dma(co, 1, buf_b, sem_b).start()
        dma(co, 0, buf_a, sem_a).wait()
        process(buf_a, co, 0)
        dma(co, 2, buf_a, sem_a).start()
        dma(co, 1, buf_b, sem_b).wait()
        process(buf_b, co, 1)
        dma(co, 3, buf_b, sem_b).start()
        dma(co, 2, buf_a, sem_a).wait()
        process(buf_a, co, 2)

        @pl.when(co + 1 < NCO)
        def _():
            dma(co + 1, 0, buf_a, sem_a).start()

        dma(co, 3, buf_b, sem_b).wait()
        process(buf_b, co, 3)
        return 0

    lax.fori_loop(0, NCO, co_body, 0, unroll=False)
    pltpu.sync_copy(obuf, out_hbm.at[b])


@jax.jit
def kernel(inputs):
    x = inputs
    x6 = jnp.transpose(x.reshape(B, S // 8, 8, 2, 128),
                       (0, 1, 3, 2, 4)).reshape(B, M, 128)
    mesh = plsc.VectorSubcoreMesh(core_axis_name="c", subcore_axis_name="s")
    run = pl.kernel(
        _sc_body,
        out_type=jax.ShapeDtypeStruct((B, C * K), jnp.float32),
        mesh=mesh,
        compiler_params=pltpu.CompilerParams(
            use_tc_tiling_on_sc=False, needs_layout_passes=False),
        scratch_types=[
            pltpu.VMEM((QR, L), jnp.float32),
            pltpu.VMEM((QR, L), jnp.float32),
            pltpu.VMEM((NCH, L), jnp.float32),
            pltpu.VMEM((2 * N2, L), jnp.float32),
            pltpu.VMEM((K, L), jnp.int32),
            pltpu.VMEM((K, L), jnp.int32),
            pltpu.VMEM((2 * K, L), jnp.float32),
            pltpu.VMEM((C * K,), jnp.float32),
            pltpu.SemaphoreType.DMA,
            pltpu.SemaphoreType.DMA,
        ],
    )
    return run(x6)
